# baseline (speedup 1.0000x reference)
import functools

import jax
import jax.numpy as jnp
from jax import lax
from jax.experimental import pallas as pl
from jax.experimental.pallas import tpu as pltpu
from jax.experimental.pallas import tpu_sc as plsc

_VOCAB = 100000
_EMBED = 128
_BATCH = 4
_SEQ = 2048
_L = 16


def _make_sc_embed(num_rows: int, embed: int, seq: int):
    info = plsc.get_sparse_core_info()
    nc, ns = info.num_cores, info.num_subcores
    nw = nc * ns
    assert num_rows % nw == 0
    rows_per_w = num_rows // nw
    assert seq % rows_per_w == 0 or rows_per_w % seq == 0
    mesh = plsc.VectorSubcoreMesh(core_axis_name="c", subcore_axis_name="s")

    nchunk = 4
    assert rows_per_w % nchunk == 0
    crows = rows_per_w // nchunk

    @functools.partial(
        pl.kernel,
        mesh=mesh,
        out_type=jax.ShapeDtypeStruct((num_rows, embed), jnp.float32),
        scratch_types=[
            pltpu.VMEM((rows_per_w,), jnp.int32),
            pltpu.VMEM((rows_per_w, embed), jnp.float32),
        ]
        + [pltpu.SemaphoreType.DMA] * (3 * nchunk),
    )
    def sc_embed(x_hbm, tok_hbm, pos_hbm, out_hbm, idx_v, rows_v, *sems):
        psem = sems[0:nchunk]
        gsem = sems[nchunk : 2 * nchunk]
        wsem = sems[2 * nchunk : 3 * nchunk]
        wid = lax.axis_index("s") * nc + lax.axis_index("c")
        base = wid * rows_per_w
        pos_base = lax.rem(base, seq)

        pos_cp, g_cp, w_cp = [None] * nchunk, [None] * nchunk, [None] * nchunk
        for c in range(nchunk):
            pos_cp[c] = pltpu.async_copy(
                pos_hbm.at[pl.ds(pos_base + c * crows, crows)],
                rows_v.at[pl.ds(c * crows, crows)],
                psem[c],
            )
        pltpu.sync_copy(x_hbm.at[pl.ds(base, rows_per_w)], idx_v)

        def fire_gather(c):
            pos_cp[c].wait()
            g_cp[c] = pltpu.async_copy(
                tok_hbm.at[idx_v.at[pl.ds(c * crows, crows)]],
                rows_v.at[pl.ds(c * crows, crows)],
                gsem[c],
                add=True,
            )

        def fire_wb(c):
            g_cp[c].wait()
            w_cp[c] = pltpu.async_copy(
                rows_v.at[pl.ds(c * crows, crows)],
                out_hbm.at[pl.ds(base + c * crows, crows)],
                wsem[c],
            )

        fire_gather(0)
        for c in range(1, nchunk):
            fire_gather(c)
            fire_wb(c - 1)
        fire_wb(nchunk - 1)
        for c in range(nchunk):
            w_cp[c].wait()

    return sc_embed


def kernel(x, token_table, pos_table):
    b, s = x.shape
    embed = token_table.shape[1]
    x_flat = x.reshape(b * s)
    fn = _make_sc_embed(b * s, embed, s)
    out = fn(x_flat, token_table, pos_table)
    return out.reshape(b, s, embed)


# speedup vs baseline: 1.0015x; 1.0015x over previous
import functools

import jax
import jax.numpy as jnp
from jax import lax
from jax.experimental import pallas as pl
from jax.experimental.pallas import tpu as pltpu
from jax.experimental.pallas import tpu_sc as plsc


def _make_sc_embed(batch: int, seq: int, embed: int):
    info = plsc.get_sparse_core_info()
    nc, ns = info.num_cores, info.num_subcores
    nw = nc * ns
    num_rows = batch * seq
    assert num_rows % nw == 0
    rows_per_w = num_rows // nw
    assert seq % rows_per_w == 0
    chunks_per_batch = seq // rows_per_w
    nchunk = 4
    assert rows_per_w % nchunk == 0
    crows = rows_per_w // nchunk
    mesh = plsc.VectorSubcoreMesh(core_axis_name="c", subcore_axis_name="s")

    @functools.partial(
        pl.kernel,
        mesh=mesh,
        out_type=jax.ShapeDtypeStruct((batch, seq, embed), jnp.float32),
        scratch_types=[
            pltpu.VMEM((rows_per_w,), jnp.int32),
            pltpu.VMEM((rows_per_w, embed), jnp.float32),
            pltpu.SemaphoreType.DMA,
        ]
        + [pltpu.SemaphoreType.DMA] * (2 * nchunk),
    )
    def sc_embed(x_hbm, tok_hbm, pos_hbm, out_hbm, idx_v, rows_v, psem, *sems):
        gsem = sems[0:nchunk]
        wsem = sems[nchunk : 2 * nchunk]
        wid = lax.axis_index("s") * nc + lax.axis_index("c")
        b = wid // chunks_per_batch
        s0 = (wid % chunks_per_batch) * rows_per_w

        pos_cp = pltpu.async_copy(pos_hbm.at[pl.ds(s0, rows_per_w)], rows_v, psem)
        pltpu.sync_copy(x_hbm.at[b, pl.ds(s0, rows_per_w)], idx_v)
        pos_cp.wait()

        g_cp = [
            pltpu.async_copy(
                tok_hbm.at[idx_v.at[pl.ds(c * crows, crows)]],
                rows_v.at[pl.ds(c * crows, crows)],
                gsem[c],
                add=True,
            )
            for c in range(nchunk)
        ]
        w_cp = []
        for c in range(nchunk):
            g_cp[c].wait()
            w_cp.append(
                pltpu.async_copy(
                    rows_v.at[pl.ds(c * crows, crows)],
                    out_hbm.at[b, pl.ds(s0 + c * crows, crows)],
                    wsem[c],
                )
            )
        for c in range(nchunk):
            w_cp[c].wait()

    return sc_embed


def kernel(x, token_table, pos_table):
    b, s = x.shape
    embed = token_table.shape[1]
    fn = _make_sc_embed(b, s, embed)
    return fn(x, token_table, pos_table)


# speedup vs baseline: 1.0131x; 1.0116x over previous
import functools

import jax
import jax.numpy as jnp
from jax import lax
from jax.experimental import pallas as pl
from jax.experimental.pallas import tpu as pltpu
from jax.experimental.pallas import tpu_sc as plsc


def _make_sc_embed(batch: int, seq: int, embed: int):
    info = plsc.get_sparse_core_info()
    nc, ns = info.num_cores, info.num_subcores
    nw = nc * ns
    num_rows = batch * seq
    assert num_rows % nw == 0
    rows_per_w = num_rows // nw
    assert seq % rows_per_w == 0
    chunks_per_batch = seq // rows_per_w
    nchunk = 4
    assert rows_per_w % nchunk == 0
    crows = rows_per_w // nchunk
    mesh = plsc.VectorSubcoreMesh(core_axis_name="c", subcore_axis_name="s")

    @functools.partial(
        pl.kernel,
        mesh=mesh,
        out_type=jax.ShapeDtypeStruct((batch, seq, embed), jnp.float32),
        scratch_types=[
            pltpu.VMEM((rows_per_w,), jnp.int32),
            pltpu.VMEM((rows_per_w, embed), jnp.float32),
            pltpu.VMEM_SHARED((4, rows_per_w, embed), jnp.float32),
            pltpu.SemaphoreType.DMA,
        ]
        + [pltpu.SemaphoreType.DMA] * (2 * nchunk),
    )
    def sc_embed(x_hbm, tok_hbm, pos_hbm, out_hbm, idx_v, rows_v, pos_sh, psem, *sems):
        gsem = sems[0:nchunk]
        wsem = sems[nchunk : 2 * nchunk]
        sid = lax.axis_index("s")
        wid = sid * nc + lax.axis_index("c")
        b = wid // chunks_per_batch
        s0 = (wid % chunks_per_batch) * rows_per_w

        @pl.when(sid < 4)
        def _leader():
            pltpu.sync_copy(pos_hbm.at[pl.ds(s0, rows_per_w)], pos_sh.at[sid])

        plsc.subcore_barrier()
        pos_cp = pltpu.async_copy(pos_sh.at[lax.rem(sid, 4)], rows_v, psem)
        pltpu.sync_copy(x_hbm.at[b, pl.ds(s0, rows_per_w)], idx_v)
        pos_cp.wait()

        g_cp = [
            pltpu.async_copy(
                tok_hbm.at[idx_v.at[pl.ds(c * crows, crows)]],
                rows_v.at[pl.ds(c * crows, crows)],
                gsem[c],
                add=True,
            )
            for c in range(nchunk)
        ]
        w_cp = []
        for c in range(nchunk):
            g_cp[c].wait()
            w_cp.append(
                pltpu.async_copy(
                    rows_v.at[pl.ds(c * crows, crows)],
                    out_hbm.at[b, pl.ds(s0 + c * crows, crows)],
                    wsem[c],
                )
            )
        for c in range(nchunk):
            w_cp[c].wait()

    return sc_embed


def kernel(x, token_table, pos_table):
    b, s = x.shape
    embed = token_table.shape[1]
    fn = _make_sc_embed(b, s, embed)
    return fn(x, token_table, pos_table)
